# baseline (speedup 1.0000x reference)
import functools

import jax
import jax.numpy as jnp
from jax import lax
from jax.experimental import pallas as pl
from jax.experimental.pallas import tpu as pltpu
from jax.experimental.pallas import tpu_sc as plsc

_f32 = jnp.float32
_KN = 16


def _pad16(c):
    return -(-c // 16) * 16


def _leaky(x):
    return jnp.where(x >= 0, x, 0.2 * x)


def _fold(p):
    W = p["W"] * p["gamma"][:, None]
    b = p["gamma"] * p["b"] + p["beta"]
    return W.T, b[None, :]


def _gather_rows(table, idx):
    N, D = table.shape
    M = idx.shape[0]
    info = plsc.get_sparse_core_info()
    nc, ns = info.num_cores, info.num_subcores
    nw = nc * ns
    assert M % nw == 0 and (M // nw) % 8 == 0, (M, nw)
    per_w = M // nw
    ch = per_w
    while ch * (D + 1) * 4 > 360000 and ch > 8:
        ch //= 2
    iters = per_w // ch
    mesh = plsc.VectorSubcoreMesh(core_axis_name="c", subcore_axis_name="s")

    def body(table_hbm, idx_hbm, out_hbm, idx_v, rows_v, sem):
        wid = lax.axis_index("s") * nc + lax.axis_index("c")
        base = wid * per_w

        def step(c, carry):
            off = base + c * ch
            pltpu.sync_copy(idx_hbm.at[pl.ds(off, ch)], idx_v)
            pltpu.async_copy(table_hbm.at[idx_v], rows_v, sem).wait()
            pltpu.sync_copy(rows_v, out_hbm.at[pl.ds(off, ch)])
            return carry

        if iters == 1:
            step(0, 0)
        else:
            lax.fori_loop(0, iters, step, 0)

    run = pl.kernel(
        body,
        out_type=jax.ShapeDtypeStruct((M, D), _f32),
        mesh=mesh,
        scratch_types=[
            pltpu.VMEM((ch,), jnp.int32),
            pltpu.VMEM((ch, D), _f32),
            pltpu.SemaphoreType.DMA,
        ],
        compiler_params=pltpu.CompilerParams(use_tc_tiling_on_sc=False),
    )
    return run(table, idx)


def _full(w):
    return pl.BlockSpec(w.shape, lambda i: (0,) * w.ndim)


def _rows(P, C):
    return pl.BlockSpec((P, C), lambda i: (i, 0))


def _idx_t(idx2d):
    N, Kn = idx2d.shape
    P = min(2048, N)

    def body(x_ref, o_ref):
        o_ref[...] = x_ref[...].T

    out = pl.pallas_call(
        body,
        grid=(N // P,),
        in_specs=[pl.BlockSpec((P, Kn), lambda i: (i, 0))],
        out_specs=pl.BlockSpec((Kn, P), lambda i: (0, i)),
        out_shape=jax.ShapeDtypeStruct((Kn, N), jnp.int32),
    )(idx2d)
    return out.reshape(-1)


def _enc_pre(feature, xyz, w1, b1, ws, bs, pre, P):
    N, Cin = feature.shape
    d2 = w1.shape[1]
    C2 = ws.shape[1]
    Dg = _pad16(d2 + 4)
    pad = Dg - d2 - 4

    def body(x_ref, xyz_ref, *refs):
        if pre is not None:
            pw_ref, pb_ref = refs[0], refs[1]
            wrefs = refs[2:6]
            t_ref, sc_ref = refs[6], refs[7]
            x = jnp.dot(x_ref[...], pw_ref[...],
                        preferred_element_type=_f32) + pb_ref[...]
        else:
            wrefs = refs[0:4]
            t_ref, sc_ref = refs[4], refs[5]
            x = x_ref[...]
        w1_ref, b1_ref, ws_ref, bs_ref = wrefs
        fpc = _leaky(jnp.dot(x, w1_ref[...], preferred_element_type=_f32)
                     + b1_ref[...])
        sc_ref[...] = jnp.dot(x, ws_ref[...],
                              preferred_element_type=_f32) + bs_ref[...]
        xyzb = xyz_ref[...]
        n2 = jnp.sum(xyzb * xyzb, axis=1, keepdims=True)
        parts = [fpc, xyzb, n2]
        if pad:
            parts.append(jnp.zeros((fpc.shape[0], pad), _f32))
        t_ref[...] = jnp.concatenate(parts, axis=1)

    ins = [feature, xyz]
    in_specs = [_rows(P, Cin), _rows(P, 3)]
    if pre is not None:
        ins += [pre[0], pre[1]]
        in_specs += [_full(pre[0]), _full(pre[1])]
    ins += [w1, b1, ws, bs]
    in_specs += [_full(w1), _full(b1), _full(ws), _full(bs)]

    return pl.pallas_call(
        body,
        grid=(N // P,),
        in_specs=in_specs,
        out_specs=[_rows(P, Dg), _rows(P, C2)],
        out_shape=[jax.ShapeDtypeStruct((N, Dg), _f32),
                   jax.ShapeDtypeStruct((N, C2), _f32)],
    )(*ins)


def _kblock(Kn, P, C):
    return pl.BlockSpec((Kn, P, C), lambda i: (0, i, 0))


def _fold16(y, op):
    a = op(y[:8], y[8:])
    a = op(a[:4], a[4:])
    a = op(a[:2], a[2:])
    return op(a[0], a[1])


def _att_agg(fcat, fcW, P, d):
    att = jnp.dot(fcat, fcW, preferred_element_type=_f32)
    e = jnp.exp(att)
    y = jnp.concatenate([e, fcat * e], axis=1).reshape(_KN, P, 2 * d)
    y0 = _fold16(y, jnp.add)
    return y0[:, d:] / y0[:, :d]


def _att_stage1(g1, xyz, w0, wtile, wga, bx1, fcW, wm, bm, wx2, bx2, d2, P):
    Kn, N, Dg = g1.shape
    d = 2 * d2
    PK = P * _KN
    d2p = _pad16(d2)

    def body(g_ref, xyz_ref, w0_ref, wt_ref, wga_ref, bx1_ref, fcw_ref,
             wm_ref, bm_ref, wx2_ref, bx2_ref, agg_ref, fxyz2_ref):
        gf = g_ref[...].reshape(PK, Dg)
        xyzb = xyz_ref[...]
        n2i = jnp.sum(xyzb * xyzb, axis=1, keepdims=True)
        bpt = jnp.dot(xyzb, wt_ref[...],
                      preferred_element_type=_f32) + bx1_ref[...]
        v4 = jnp.concatenate([-2.0 * xyzb, jnp.ones((P, 1), _f32)], axis=1)
        u4 = gf[:, d2:d2 + 4]
        m = u4 * jnp.broadcast_to(v4[None], (Kn, P, 4)).reshape(PK, 4)
        dis2 = (jnp.dot(m, jnp.ones((4, 1), _f32),
                        preferred_element_type=_f32)
                + jnp.broadcast_to(n2i[None], (Kn, P, 1)).reshape(PK, 1))
        dis = jnp.sqrt(jnp.maximum(dis2, 0.0) + 1e-12)
        f_xyz = _leaky(dis * w0_ref[...]
                       + jnp.dot(gf, wga_ref[...], preferred_element_type=_f32)
                       + jnp.broadcast_to(bpt[None], (Kn, P, d2)
                                          ).reshape(PK, d2))
        fcat = jnp.concatenate([gf[:, :d2], f_xyz], axis=1)
        agg = _att_agg(fcat, fcw_ref[...], P, d)
        f_agg = _leaky(jnp.dot(agg, wm_ref[...],
                               preferred_element_type=_f32) + bm_ref[...])
        if d2p > d2:
            f_agg = jnp.concatenate(
                [f_agg, jnp.zeros((P, d2p - d2), _f32)], axis=1)
        agg_ref[...] = f_agg
        fxyz2_ref[...] = _leaky(jnp.dot(f_xyz, wx2_ref[...],
                                        preferred_element_type=_f32)
                                + bx2_ref[...]).reshape(Kn, P, d2)

    return pl.pallas_call(
        body,
        grid=(N // P,),
        in_specs=[_kblock(Kn, P, Dg), _rows(P, 3), _full(w0), _full(wtile),
                  _full(wga), _full(bx1), _full(fcW), _full(wm), _full(bm),
                  _full(wx2), _full(bx2)],
        out_specs=[_rows(P, d2p), _kblock(Kn, P, d2)],
        out_shape=[jax.ShapeDtypeStruct((N, d2p), _f32),
                   jax.ShapeDtypeStruct((Kn, N, d2), _f32)],
    )(g1, xyz, w0, wtile, wga, bx1, fcW, wm, bm, wx2, bx2)


def _att_stage2(g2, fxyz2, fcW, wm, bm, w2, b2, sc, d2, P):
    N = sc.shape[0]
    d = 2 * d2
    C2 = sc.shape[1]
    PK = P * _KN
    Kn = g2.shape[0]
    d2p = g2.shape[2]

    def body(g_ref, fx_ref, fcw_ref, wm_ref, bm_ref, w2_ref, b2_ref,
             sc_ref, o_ref):
        f_neigh = g_ref[...][:, :, :d2].reshape(PK, d2)
        fcat = jnp.concatenate([f_neigh, fx_ref[...].reshape(PK, d2)], axis=1)
        agg = _att_agg(fcat, fcw_ref[...], P, d)
        f = _leaky(jnp.dot(agg, wm_ref[...],
                           preferred_element_type=_f32) + bm_ref[...])
        fpc = jnp.dot(f, w2_ref[...], preferred_element_type=_f32) + b2_ref[...]
        o_ref[...] = _leaky(fpc + sc_ref[...])

    return pl.pallas_call(
        body,
        grid=(N // P,),
        in_specs=[_kblock(Kn, P, d2p), _kblock(Kn, P, d2), _full(fcW),
                  _full(wm), _full(bm), _full(w2), _full(b2), _rows(P, C2)],
        out_specs=_rows(P, C2),
        out_shape=jax.ShapeDtypeStruct((N, C2), _f32),
    )(g2, fxyz2, fcW, wm, bm, w2, b2, sc)


def _pool_max(g3, P):
    Kn, N2, C = g3.shape

    def body(g_ref, o_ref):
        o_ref[...] = _fold16(g_ref[...], jnp.maximum)

    return pl.pallas_call(
        body,
        grid=(N2 // P,),
        in_specs=[_kblock(Kn, P, C)],
        out_specs=_rows(P, C),
        out_shape=jax.ShapeDtypeStruct((N2, C), _f32),
    )(g3)


def _conv(x, w, b, P):
    N, Cin = x.shape
    Cout = w.shape[1]

    def body(x_ref, w_ref, b_ref, o_ref):
        o_ref[...] = _leaky(jnp.dot(x_ref[...], w_ref[...],
                                    preferred_element_type=_f32) + b_ref[...])

    return pl.pallas_call(
        body,
        grid=(N // P,),
        in_specs=[_rows(P, Cin), _full(w), _full(b)],
        out_specs=_rows(P, Cout),
        out_shape=jax.ShapeDtypeStruct((N, Cout), _f32),
    )(x, w, b)


def _dec_step(enc, itp, w1, w2, b, P):
    N, C1 = enc.shape
    C2 = itp.shape[1]
    Cout = w1.shape[1]

    def body(e_ref, i_ref, w1_ref, w2_ref, b_ref, o_ref):
        y = jnp.dot(e_ref[...], w1_ref[...], preferred_element_type=_f32)
        y = y + jnp.dot(i_ref[...], w2_ref[...], preferred_element_type=_f32)
        o_ref[...] = _leaky(y + b_ref[...])

    return pl.pallas_call(
        body,
        grid=(N // P,),
        in_specs=[_rows(P, C1), _rows(P, C2), _full(w1), _full(w2), _full(b)],
        out_specs=_rows(P, Cout),
        out_shape=jax.ShapeDtypeStruct((N, Cout), _f32),
    )(enc, itp, w1, w2, b)


def _fc_head(x, w1, b1, w2, b2, w3, b3, P):
    N = x.shape[0]
    Cout = w3.shape[1]

    def body(x_ref, w1r, b1r, w2r, b2r, w3r, b3r, o_ref):
        h = _leaky(jnp.dot(x_ref[...], w1r[...],
                           preferred_element_type=_f32) + b1r[...])
        h = _leaky(jnp.dot(h, w2r[...], preferred_element_type=_f32) + b2r[...])
        y = jnp.dot(h, w3r[...], preferred_element_type=_f32) + b3r[...]
        o_ref[...] = y.T

    return pl.pallas_call(
        body,
        grid=(N // P,),
        in_specs=[_rows(P, x.shape[1]), _full(w1), _full(b1), _full(w2),
                  _full(b2), _full(w3), _full(b3)],
        out_specs=pl.BlockSpec((Cout, P), lambda i: (0, i)),
        out_shape=jax.ShapeDtypeStruct((Cout, N), _f32),
    )(x, w1, b1, w2, b2, w3, b3)


_P_ATT = [256, 128, 128, 64]


def kernel(features, xyz_0, xyz_1, xyz_2, xyz_3, neigh_idx_0, neigh_idx_1,
           neigh_idx_2, neigh_idx_3, sub_idx_0, sub_idx_1, sub_idx_2,
           sub_idx_3, interp_idx_0, interp_idx_1, interp_idx_2, interp_idx_3,
           params):
    xyzs = [x[0] for x in (xyz_0, xyz_1, xyz_2, xyz_3)]
    neighs = [_idx_t(n[0]) for n in
              (neigh_idx_0, neigh_idx_1, neigh_idx_2, neigh_idx_3)]
    subs = [_idx_t(s[0]) for s in
            (sub_idx_0, sub_idx_1, sub_idx_2, sub_idx_3)]
    interps = [t[0, :, 0] for t in
               (interp_idx_0, interp_idx_1, interp_idx_2, interp_idx_3)]

    g0 = params["bn0_gamma"]
    preW = params["fc0W"] * g0[None, :]
    preb = (params["fc0b"] * g0 + params["bn0_beta"])[None, :]

    feature = features[0]
    enc_feats = []
    for i in range(4):
        p = params["enc"][i]
        w1, b1 = _fold(p["mlp1"])
        ws, bs = _fold(p["shortcut"])
        wx1, bx1 = _fold(p["xyz1"])
        wx2, bx2 = _fold(p["xyz2"])
        d2w = w1.shape[1]
        dgw = _pad16(d2w + 4)
        w0 = wx1[0:1]
        wtile = wx1[1:4] + wx1[4:7]
        wga = jnp.zeros((dgw, d2w), _f32).at[d2w:d2w + 3].set(
            wx1[7:10] - wx1[1:4])
        wm1, bm1 = _fold(p["att1"]["mlp"])
        wm2, bm2 = _fold(p["att2"]["mlp"])
        w2, b2 = _fold(p["mlp2"])
        d2 = w1.shape[1]
        P = _P_ATT[i]

        N = feature.shape[0] if i else features.shape[1]
        T, sc = _enc_pre(feature, xyzs[i], w1, b1, ws, bs,
                         (preW, preb) if i == 0 else None, P=256)
        G1 = _gather_rows(T, neighs[i]).reshape(_KN, N, T.shape[1])
        aggT, fxyz2 = _att_stage1(G1, xyzs[i], w0, wtile, wga, bx1,
                                  p["att1"]["fcW"], wm1, bm1, wx2, bx2, d2, P)
        G2 = _gather_rows(aggT, neighs[i]).reshape(_KN, N, aggT.shape[1])
        f_enc = _att_stage2(G2, fxyz2, p["att2"]["fcW"], wm2, bm2, w2, b2,
                            sc, d2, P)
        if i == 0:
            enc_feats.append(f_enc)
        N2 = subs[i].shape[0] // _KN
        Gs = _gather_rows(f_enc, subs[i]).reshape(_KN, N2, f_enc.shape[1])
        f_s = _pool_max(Gs, min(256, N2))
        enc_feats.append(f_s)
        feature = f_s

    wd0, bd0 = _fold(params["decoder_0"])
    feature = _conv(feature, wd0, bd0, min(256, feature.shape[0]))
    for j in range(4):
        Gi = _gather_rows(feature, interps[3 - j])
        enc = enc_feats[-j - 2]
        wj, bj = _fold(params["dec"][j])
        c1 = enc.shape[1]
        feature = _dec_step(enc, Gi, wj[:c1], wj[c1:], bj, 256)

    w_fc1, b_fc1 = _fold(params["fc1"])
    w_fc2, b_fc2 = _fold(params["fc2"])
    out = _fc_head(feature, w_fc1, b_fc1, w_fc2, b_fc2,
                   params["fc3W"].T, params["fc3b"][None, :], 512)
    return out[None]


# speedup vs baseline: 1.0001x; 1.0001x over previous
import functools

import jax
import jax.numpy as jnp
from jax import lax
from jax.experimental import pallas as pl
from jax.experimental.pallas import tpu as pltpu
from jax.experimental.pallas import tpu_sc as plsc

_f32 = jnp.float32
_KN = 16


def _pad16(c):
    return -(-c // 16) * 16


def _leaky(x):
    return jnp.where(x >= 0, x, 0.2 * x)


def _fold(p):
    W = p["W"] * p["gamma"][:, None]
    b = p["gamma"] * p["b"] + p["beta"]
    return W.T, b[None, :]


def _gather_rows(table, idx, slabs=None):
    N, D = table.shape
    M = idx.shape[0]
    info = plsc.get_sparse_core_info()
    nc, ns = info.num_cores, info.num_subcores
    nw = nc * ns
    assert M % nw == 0 and (M // nw) % 8 == 0, (M, nw)
    per_w = M // nw
    ch = per_w
    while ch * (D + 1) * 4 > 360000 and ch > 8:
        ch //= 2
    iters = per_w // ch
    rows = M if slabs is None else M // slabs
    out_t = jax.ShapeDtypeStruct((M, D) if slabs is None
                                 else (slabs, rows, D), _f32)
    mesh = plsc.VectorSubcoreMesh(core_axis_name="c", subcore_axis_name="s")

    def body(table_hbm, idx_hbm, out_hbm, idx_v, rows_v, sem):
        wid = lax.axis_index("s") * nc + lax.axis_index("c")
        base = wid * per_w

        def step(c, carry):
            off = base + c * ch
            pltpu.sync_copy(idx_hbm.at[pl.ds(off, ch)], idx_v)
            pltpu.async_copy(table_hbm.at[idx_v], rows_v, sem).wait()
            if slabs is None:
                pltpu.sync_copy(rows_v, out_hbm.at[pl.ds(off, ch)])
            else:
                pltpu.sync_copy(
                    rows_v, out_hbm.at[off // rows, pl.ds(off % rows, ch)])
            return carry

        if iters == 1:
            step(0, 0)
        else:
            lax.fori_loop(0, iters, step, 0)

    run = pl.kernel(
        body,
        out_type=out_t,
        mesh=mesh,
        scratch_types=[
            pltpu.VMEM((ch,), jnp.int32),
            pltpu.VMEM((ch, D), _f32),
            pltpu.SemaphoreType.DMA,
        ],
        compiler_params=pltpu.CompilerParams(use_tc_tiling_on_sc=False),
    )
    return run(table, idx)


def _full(w):
    return pl.BlockSpec(w.shape, lambda i: (0,) * w.ndim)


def _rows(P, C):
    return pl.BlockSpec((P, C), lambda i: (i, 0))


def _idx_t(idx2d):
    N, Kn = idx2d.shape
    P = min(2048, N)

    def body(x_ref, o_ref):
        o_ref[...] = x_ref[...].T

    out = pl.pallas_call(
        body,
        grid=(N // P,),
        in_specs=[pl.BlockSpec((P, Kn), lambda i: (i, 0))],
        out_specs=pl.BlockSpec((Kn, P), lambda i: (0, i)),
        out_shape=jax.ShapeDtypeStruct((Kn, N), jnp.int32),
    )(idx2d)
    return out.reshape(-1)


def _enc_pre(feature, xyz, w1, b1, ws, bs, pre, P):
    N, Cin = feature.shape
    d2 = w1.shape[1]
    C2 = ws.shape[1]
    Dg = _pad16(d2 + 4)
    pad = Dg - d2 - 4

    def body(x_ref, xyz_ref, *refs):
        if pre is not None:
            pw_ref, pb_ref = refs[0], refs[1]
            wrefs = refs[2:6]
            t_ref, sc_ref = refs[6], refs[7]
            x = jnp.dot(x_ref[...], pw_ref[...],
                        preferred_element_type=_f32) + pb_ref[...]
        else:
            wrefs = refs[0:4]
            t_ref, sc_ref = refs[4], refs[5]
            x = x_ref[...]
        w1_ref, b1_ref, ws_ref, bs_ref = wrefs
        fpc = _leaky(jnp.dot(x, w1_ref[...], preferred_element_type=_f32)
                     + b1_ref[...])
        sc_ref[...] = jnp.dot(x, ws_ref[...],
                              preferred_element_type=_f32) + bs_ref[...]
        xyzb = xyz_ref[...]
        n2 = jnp.sum(xyzb * xyzb, axis=1, keepdims=True)
        parts = [fpc, xyzb, n2]
        if pad:
            parts.append(jnp.zeros((fpc.shape[0], pad), _f32))
        t_ref[...] = jnp.concatenate(parts, axis=1)

    ins = [feature, xyz]
    in_specs = [_rows(P, Cin), _rows(P, 3)]
    if pre is not None:
        ins += [pre[0], pre[1]]
        in_specs += [_full(pre[0]), _full(pre[1])]
    ins += [w1, b1, ws, bs]
    in_specs += [_full(w1), _full(b1), _full(ws), _full(bs)]

    return pl.pallas_call(
        body,
        grid=(N // P,),
        in_specs=in_specs,
        out_specs=[_rows(P, Dg), _rows(P, C2)],
        out_shape=[jax.ShapeDtypeStruct((N, Dg), _f32),
                   jax.ShapeDtypeStruct((N, C2), _f32)],
    )(*ins)


def _kblock(Kn, P, C):
    return pl.BlockSpec((Kn, P, C), lambda i: (0, i, 0))


def _fold16(y, op):
    a = op(y[:8], y[8:])
    a = op(a[:4], a[4:])
    a = op(a[:2], a[2:])
    return op(a[0], a[1])


def _att_agg(fcat, fcW, P, d):
    att = jnp.dot(fcat, fcW, preferred_element_type=_f32)
    e = jnp.exp(att)
    y = jnp.concatenate([e, fcat * e], axis=1).reshape(_KN, P, 2 * d)
    y0 = _fold16(y, jnp.add)
    return y0[:, d:] / y0[:, :d]


def _att_stage1(g1, xyz, w0, wtile, wga, bx1, fcW, wm, bm, wx2, bx2, d2, P):
    Kn, N, Dg = g1.shape
    d = 2 * d2
    PK = P * _KN
    d2p = _pad16(d2)

    def body(g_ref, xyz_ref, w0_ref, wt_ref, wga_ref, bx1_ref, fcw_ref,
             wm_ref, bm_ref, wx2_ref, bx2_ref, agg_ref, fxyz2_ref):
        gf = g_ref[...].reshape(PK, Dg)
        xyzb = xyz_ref[...]
        n2i = jnp.sum(xyzb * xyzb, axis=1, keepdims=True)
        bpt = jnp.dot(xyzb, wt_ref[...],
                      preferred_element_type=_f32) + bx1_ref[...]
        v4 = jnp.concatenate([-2.0 * xyzb, jnp.ones((P, 1), _f32)], axis=1)
        u4 = gf[:, d2:d2 + 4]
        m = u4 * jnp.broadcast_to(v4[None], (Kn, P, 4)).reshape(PK, 4)
        dis2 = (jnp.dot(m, jnp.ones((4, 1), _f32),
                        preferred_element_type=_f32)
                + jnp.broadcast_to(n2i[None], (Kn, P, 1)).reshape(PK, 1))
        dis = jnp.sqrt(jnp.maximum(dis2, 0.0) + 1e-12)
        f_xyz = _leaky(dis * w0_ref[...]
                       + jnp.dot(gf, wga_ref[...], preferred_element_type=_f32)
                       + jnp.broadcast_to(bpt[None], (Kn, P, d2)
                                          ).reshape(PK, d2))
        fcat = jnp.concatenate([gf[:, :d2], f_xyz], axis=1)
        agg = _att_agg(fcat, fcw_ref[...], P, d)
        f_agg = _leaky(jnp.dot(agg, wm_ref[...],
                               preferred_element_type=_f32) + bm_ref[...])
        if d2p > d2:
            f_agg = jnp.concatenate(
                [f_agg, jnp.zeros((P, d2p - d2), _f32)], axis=1)
        agg_ref[...] = f_agg
        fxyz2_ref[...] = _leaky(jnp.dot(f_xyz, wx2_ref[...],
                                        preferred_element_type=_f32)
                                + bx2_ref[...]).reshape(Kn, P, d2)

    return pl.pallas_call(
        body,
        grid=(N // P,),
        in_specs=[_kblock(Kn, P, Dg), _rows(P, 3), _full(w0), _full(wtile),
                  _full(wga), _full(bx1), _full(fcW), _full(wm), _full(bm),
                  _full(wx2), _full(bx2)],
        out_specs=[_rows(P, d2p), _kblock(Kn, P, d2)],
        out_shape=[jax.ShapeDtypeStruct((N, d2p), _f32),
                   jax.ShapeDtypeStruct((Kn, N, d2), _f32)],
    )(g1, xyz, w0, wtile, wga, bx1, fcW, wm, bm, wx2, bx2)


def _att_stage2(g2, fxyz2, fcW, wm, bm, w2, b2, sc, d2, P):
    N = sc.shape[0]
    d = 2 * d2
    C2 = sc.shape[1]
    PK = P * _KN
    Kn = g2.shape[0]
    d2p = g2.shape[2]

    def body(g_ref, fx_ref, fcw_ref, wm_ref, bm_ref, w2_ref, b2_ref,
             sc_ref, o_ref):
        f_neigh = g_ref[...][:, :, :d2].reshape(PK, d2)
        fcat = jnp.concatenate([f_neigh, fx_ref[...].reshape(PK, d2)], axis=1)
        agg = _att_agg(fcat, fcw_ref[...], P, d)
        f = _leaky(jnp.dot(agg, wm_ref[...],
                           preferred_element_type=_f32) + bm_ref[...])
        fpc = jnp.dot(f, w2_ref[...], preferred_element_type=_f32) + b2_ref[...]
        o_ref[...] = _leaky(fpc + sc_ref[...])

    return pl.pallas_call(
        body,
        grid=(N // P,),
        in_specs=[_kblock(Kn, P, d2p), _kblock(Kn, P, d2), _full(fcW),
                  _full(wm), _full(bm), _full(w2), _full(b2), _rows(P, C2)],
        out_specs=_rows(P, C2),
        out_shape=jax.ShapeDtypeStruct((N, C2), _f32),
    )(g2, fxyz2, fcW, wm, bm, w2, b2, sc)


def _pool_max(g3, P):
    Kn, N2, C = g3.shape

    def body(g_ref, o_ref):
        o_ref[...] = _fold16(g_ref[...], jnp.maximum)

    return pl.pallas_call(
        body,
        grid=(N2 // P,),
        in_specs=[_kblock(Kn, P, C)],
        out_specs=_rows(P, C),
        out_shape=jax.ShapeDtypeStruct((N2, C), _f32),
    )(g3)


def _conv(x, w, b, P):
    N, Cin = x.shape
    Cout = w.shape[1]

    def body(x_ref, w_ref, b_ref, o_ref):
        o_ref[...] = _leaky(jnp.dot(x_ref[...], w_ref[...],
                                    preferred_element_type=_f32) + b_ref[...])

    return pl.pallas_call(
        body,
        grid=(N // P,),
        in_specs=[_rows(P, Cin), _full(w), _full(b)],
        out_specs=_rows(P, Cout),
        out_shape=jax.ShapeDtypeStruct((N, Cout), _f32),
    )(x, w, b)


def _dec_step(enc, itp, w1, w2, b, P):
    N, C1 = enc.shape
    C2 = itp.shape[1]
    Cout = w1.shape[1]

    def body(e_ref, i_ref, w1_ref, w2_ref, b_ref, o_ref):
        y = jnp.dot(e_ref[...], w1_ref[...], preferred_element_type=_f32)
        y = y + jnp.dot(i_ref[...], w2_ref[...], preferred_element_type=_f32)
        o_ref[...] = _leaky(y + b_ref[...])

    return pl.pallas_call(
        body,
        grid=(N // P,),
        in_specs=[_rows(P, C1), _rows(P, C2), _full(w1), _full(w2), _full(b)],
        out_specs=_rows(P, Cout),
        out_shape=jax.ShapeDtypeStruct((N, Cout), _f32),
    )(enc, itp, w1, w2, b)


def _fc_head(x, w1, b1, w2, b2, w3, b3, P):
    N = x.shape[0]
    Cout = w3.shape[1]

    def body(x_ref, w1r, b1r, w2r, b2r, w3r, b3r, o_ref):
        h = _leaky(jnp.dot(x_ref[...], w1r[...],
                           preferred_element_type=_f32) + b1r[...])
        h = _leaky(jnp.dot(h, w2r[...], preferred_element_type=_f32) + b2r[...])
        y = jnp.dot(h, w3r[...], preferred_element_type=_f32) + b3r[...]
        o_ref[...] = y.T

    return pl.pallas_call(
        body,
        grid=(N // P,),
        in_specs=[_rows(P, x.shape[1]), _full(w1), _full(b1), _full(w2),
                  _full(b2), _full(w3), _full(b3)],
        out_specs=pl.BlockSpec((Cout, P), lambda i: (0, i)),
        out_shape=jax.ShapeDtypeStruct((Cout, N), _f32),
    )(x, w1, b1, w2, b2, w3, b3)


_P_ATT = [256, 128, 128, 64]


def kernel(features, xyz_0, xyz_1, xyz_2, xyz_3, neigh_idx_0, neigh_idx_1,
           neigh_idx_2, neigh_idx_3, sub_idx_0, sub_idx_1, sub_idx_2,
           sub_idx_3, interp_idx_0, interp_idx_1, interp_idx_2, interp_idx_3,
           params):
    xyzs = [x[0] for x in (xyz_0, xyz_1, xyz_2, xyz_3)]
    neighs = [_idx_t(n[0]) for n in
              (neigh_idx_0, neigh_idx_1, neigh_idx_2, neigh_idx_3)]
    subs = [_idx_t(s[0]) for s in
            (sub_idx_0, sub_idx_1, sub_idx_2, sub_idx_3)]
    interps = [t[0, :, 0] for t in
               (interp_idx_0, interp_idx_1, interp_idx_2, interp_idx_3)]

    g0 = params["bn0_gamma"]
    preW = params["fc0W"] * g0[None, :]
    preb = (params["fc0b"] * g0 + params["bn0_beta"])[None, :]

    feature = features[0]
    enc_feats = []
    for i in range(4):
        p = params["enc"][i]
        w1, b1 = _fold(p["mlp1"])
        ws, bs = _fold(p["shortcut"])
        wx1, bx1 = _fold(p["xyz1"])
        wx2, bx2 = _fold(p["xyz2"])
        d2w = w1.shape[1]
        dgw = _pad16(d2w + 4)
        w0 = wx1[0:1]
        wtile = wx1[1:4] + wx1[4:7]
        wga = jnp.zeros((dgw, d2w), _f32).at[d2w:d2w + 3].set(
            wx1[7:10] - wx1[1:4])
        wm1, bm1 = _fold(p["att1"]["mlp"])
        wm2, bm2 = _fold(p["att2"]["mlp"])
        w2, b2 = _fold(p["mlp2"])
        d2 = w1.shape[1]
        P = _P_ATT[i]

        N = feature.shape[0] if i else features.shape[1]
        T, sc = _enc_pre(feature, xyzs[i], w1, b1, ws, bs,
                         (preW, preb) if i == 0 else None, P=256)
        G1 = _gather_rows(T, neighs[i], slabs=_KN)
        aggT, fxyz2 = _att_stage1(G1, xyzs[i], w0, wtile, wga, bx1,
                                  p["att1"]["fcW"], wm1, bm1, wx2, bx2, d2, P)
        G2 = _gather_rows(aggT, neighs[i], slabs=_KN)
        f_enc = _att_stage2(G2, fxyz2, p["att2"]["fcW"], wm2, bm2, w2, b2,
                            sc, d2, P)
        if i == 0:
            enc_feats.append(f_enc)
        Gs = _gather_rows(f_enc, subs[i], slabs=_KN)
        f_s = _pool_max(Gs, min(256, Gs.shape[1]))
        enc_feats.append(f_s)
        feature = f_s

    wd0, bd0 = _fold(params["decoder_0"])
    feature = _conv(feature, wd0, bd0, min(256, feature.shape[0]))
    for j in range(4):
        Gi = _gather_rows(feature, interps[3 - j])
        enc = enc_feats[-j - 2]
        wj, bj = _fold(params["dec"][j])
        c1 = enc.shape[1]
        feature = _dec_step(enc, Gi, wj[:c1], wj[c1:], bj, 256)

    w_fc1, b_fc1 = _fold(params["fc1"])
    w_fc2, b_fc2 = _fold(params["fc2"])
    out = _fc_head(feature, w_fc1, b_fc1, w_fc2, b_fc2,
                   params["fc3W"].T, params["fc3b"][None, :], 512)
    return out[None]


# speedup vs baseline: 1.0154x; 1.0153x over previous
import functools

import jax
import jax.numpy as jnp
from jax import lax
from jax.experimental import pallas as pl
from jax.experimental.pallas import tpu as pltpu
from jax.experimental.pallas import tpu_sc as plsc

_f32 = jnp.float32
_KN = 16


def _pad16(c):
    return -(-c // 16) * 16


def _leaky(x):
    return jnp.where(x >= 0, x, 0.2 * x)


def _fold(p):
    W = p["W"] * p["gamma"][:, None]
    b = p["gamma"] * p["b"] + p["beta"]
    return W.T, b[None, :]


def _gather_rows(table, idx, slabs=None):
    N, D = table.shape
    M = idx.shape[0]
    info = plsc.get_sparse_core_info()
    nc, ns = info.num_cores, info.num_subcores
    nw = nc * ns
    assert M % nw == 0 and (M // nw) % 8 == 0, (M, nw)
    per_w = M // nw
    ch = per_w
    while ch * (D + 1) * 4 > 360000 and ch > 8:
        ch //= 2
    iters = per_w // ch
    rows = M if slabs is None else M // slabs
    out_t = jax.ShapeDtypeStruct((M, D) if slabs is None
                                 else (slabs, rows, D), _f32)
    mesh = plsc.VectorSubcoreMesh(core_axis_name="c", subcore_axis_name="s")

    def body(table_hbm, idx_hbm, out_hbm, idx_v, rows_v, sem):
        wid = lax.axis_index("s") * nc + lax.axis_index("c")
        base = wid * per_w

        def step(c, carry):
            off = base + c * ch
            pltpu.sync_copy(idx_hbm.at[pl.ds(off, ch)], idx_v)
            pltpu.async_copy(table_hbm.at[idx_v], rows_v, sem).wait()
            if slabs is None:
                pltpu.sync_copy(rows_v, out_hbm.at[pl.ds(off, ch)])
            else:
                pltpu.sync_copy(
                    rows_v, out_hbm.at[off // rows, pl.ds(off % rows, ch)])
            return carry

        if iters == 1:
            step(0, 0)
        else:
            lax.fori_loop(0, iters, step, 0)

    run = pl.kernel(
        body,
        out_type=out_t,
        mesh=mesh,
        scratch_types=[
            pltpu.VMEM((ch,), jnp.int32),
            pltpu.VMEM((ch, D), _f32),
            pltpu.SemaphoreType.DMA,
        ],
        compiler_params=pltpu.CompilerParams(use_tc_tiling_on_sc=False),
    )
    return run(table, idx)


def _full(w):
    return pl.BlockSpec(w.shape, lambda i: (0,) * w.ndim)


def _rows(P, C):
    return pl.BlockSpec((P, C), lambda i: (i, 0))


def _enc_pre(feature, xyz, w1, b1, ws, bs, pre, P):
    N, Cin = feature.shape
    d2 = w1.shape[1]
    C2 = ws.shape[1]
    Dg = _pad16(d2 + 4)
    pad = Dg - d2 - 4

    def body(x_ref, xyz_ref, *refs):
        if pre is not None:
            pw_ref, pb_ref = refs[0], refs[1]
            wrefs = refs[2:6]
            t_ref, sc_ref = refs[6], refs[7]
            x = jnp.dot(x_ref[...], pw_ref[...],
                        preferred_element_type=_f32) + pb_ref[...]
        else:
            wrefs = refs[0:4]
            t_ref, sc_ref = refs[4], refs[5]
            x = x_ref[...]
        w1_ref, b1_ref, ws_ref, bs_ref = wrefs
        fpc = _leaky(jnp.dot(x, w1_ref[...], preferred_element_type=_f32)
                     + b1_ref[...])
        sc_ref[...] = jnp.dot(x, ws_ref[...],
                              preferred_element_type=_f32) + bs_ref[...]
        xyzb = xyz_ref[...]
        n2 = jnp.sum(xyzb * xyzb, axis=1, keepdims=True)
        parts = [fpc, xyzb, n2]
        if pad:
            parts.append(jnp.zeros((fpc.shape[0], pad), _f32))
        t_ref[...] = jnp.concatenate(parts, axis=1)

    ins = [feature, xyz]
    in_specs = [_rows(P, Cin), _rows(P, 3)]
    if pre is not None:
        ins += [pre[0], pre[1]]
        in_specs += [_full(pre[0]), _full(pre[1])]
    ins += [w1, b1, ws, bs]
    in_specs += [_full(w1), _full(b1), _full(ws), _full(bs)]

    return pl.pallas_call(
        body,
        grid=(N // P,),
        in_specs=in_specs,
        out_specs=[_rows(P, Dg), _rows(P, C2)],
        out_shape=[jax.ShapeDtypeStruct((N, Dg), _f32),
                   jax.ShapeDtypeStruct((N, C2), _f32)],
    )(*ins)


def _kblock(Kn, P, C):
    return pl.BlockSpec((Kn, P, C), lambda i: (0, i, 0))


def _fold16(y, op):
    a = op(y[:8], y[8:])
    a = op(a[:4], a[4:])
    a = op(a[:2], a[2:])
    return op(a[0], a[1])


def _att_agg(fcat, fcW, P, d):
    att = jnp.dot(fcat, fcW, preferred_element_type=_f32)
    e = jnp.exp(att)
    y = jnp.concatenate([e, fcat * e], axis=1).reshape(_KN, P, 2 * d)
    y0 = _fold16(y, jnp.add)
    return y0[:, d:] / y0[:, :d]


def _att_stage1(g1, xyz, w0, wtile, wga, bx1, fcW, wm, bm, wx2, bx2, d2, P):
    Kn, N, Dg = g1.shape
    d = 2 * d2
    PK = P * _KN
    d2p = _pad16(d2)

    def body(g_ref, xyz_ref, w0_ref, wt_ref, wga_ref, bx1_ref, fcw_ref,
             wm_ref, bm_ref, wx2_ref, bx2_ref, agg_ref, fxyz2_ref):
        gf = g_ref[...].reshape(PK, Dg)
        xyzb = xyz_ref[...]
        n2i = jnp.sum(xyzb * xyzb, axis=1, keepdims=True)
        bpt = jnp.dot(xyzb, wt_ref[...],
                      preferred_element_type=_f32) + bx1_ref[...]
        v4 = jnp.concatenate([-2.0 * xyzb, jnp.ones((P, 1), _f32)], axis=1)
        u4 = gf[:, d2:d2 + 4]
        m = u4 * jnp.broadcast_to(v4[None], (Kn, P, 4)).reshape(PK, 4)
        dis2 = (jnp.dot(m, jnp.ones((4, 1), _f32),
                        preferred_element_type=_f32)
                + jnp.broadcast_to(n2i[None], (Kn, P, 1)).reshape(PK, 1))
        dis = jnp.sqrt(jnp.maximum(dis2, 0.0) + 1e-12)
        f_xyz = _leaky(dis * w0_ref[...]
                       + jnp.dot(gf, wga_ref[...], preferred_element_type=_f32)
                       + jnp.broadcast_to(bpt[None], (Kn, P, d2)
                                          ).reshape(PK, d2))
        fcat = jnp.concatenate([gf[:, :d2], f_xyz], axis=1)
        agg = _att_agg(fcat, fcw_ref[...], P, d)
        f_agg = _leaky(jnp.dot(agg, wm_ref[...],
                               preferred_element_type=_f32) + bm_ref[...])
        if d2p > d2:
            f_agg = jnp.concatenate(
                [f_agg, jnp.zeros((P, d2p - d2), _f32)], axis=1)
        agg_ref[...] = f_agg
        fxyz2_ref[...] = _leaky(jnp.dot(f_xyz, wx2_ref[...],
                                        preferred_element_type=_f32)
                                + bx2_ref[...]).reshape(Kn, P, d2)

    return pl.pallas_call(
        body,
        grid=(N // P,),
        in_specs=[_kblock(Kn, P, Dg), _rows(P, 3), _full(w0), _full(wtile),
                  _full(wga), _full(bx1), _full(fcW), _full(wm), _full(bm),
                  _full(wx2), _full(bx2)],
        out_specs=[_rows(P, d2p), _kblock(Kn, P, d2)],
        out_shape=[jax.ShapeDtypeStruct((N, d2p), _f32),
                   jax.ShapeDtypeStruct((Kn, N, d2), _f32)],
    )(g1, xyz, w0, wtile, wga, bx1, fcW, wm, bm, wx2, bx2)


def _att_stage2(g2, fxyz2, fcW, wm, bm, w2, b2, sc, d2, P):
    N = sc.shape[0]
    d = 2 * d2
    C2 = sc.shape[1]
    PK = P * _KN
    Kn = g2.shape[0]
    d2p = g2.shape[2]

    def body(g_ref, fx_ref, fcw_ref, wm_ref, bm_ref, w2_ref, b2_ref,
             sc_ref, o_ref):
        f_neigh = g_ref[...][:, :, :d2].reshape(PK, d2)
        fcat = jnp.concatenate([f_neigh, fx_ref[...].reshape(PK, d2)], axis=1)
        agg = _att_agg(fcat, fcw_ref[...], P, d)
        f = _leaky(jnp.dot(agg, wm_ref[...],
                           preferred_element_type=_f32) + bm_ref[...])
        fpc = jnp.dot(f, w2_ref[...], preferred_element_type=_f32) + b2_ref[...]
        o_ref[...] = _leaky(fpc + sc_ref[...])

    return pl.pallas_call(
        body,
        grid=(N // P,),
        in_specs=[_kblock(Kn, P, d2p), _kblock(Kn, P, d2), _full(fcW),
                  _full(wm), _full(bm), _full(w2), _full(b2), _rows(P, C2)],
        out_specs=_rows(P, C2),
        out_shape=jax.ShapeDtypeStruct((N, C2), _f32),
    )(g2, fxyz2, fcW, wm, bm, w2, b2, sc)


def _pool_max(g3, P):
    Kn, N2, C = g3.shape

    def body(g_ref, o_ref):
        o_ref[...] = _fold16(g_ref[...], jnp.maximum)

    return pl.pallas_call(
        body,
        grid=(N2 // P,),
        in_specs=[_kblock(Kn, P, C)],
        out_specs=_rows(P, C),
        out_shape=jax.ShapeDtypeStruct((N2, C), _f32),
    )(g3)


def _conv(x, w, b, P):
    N, Cin = x.shape
    Cout = w.shape[1]

    def body(x_ref, w_ref, b_ref, o_ref):
        o_ref[...] = _leaky(jnp.dot(x_ref[...], w_ref[...],
                                    preferred_element_type=_f32) + b_ref[...])

    return pl.pallas_call(
        body,
        grid=(N // P,),
        in_specs=[_rows(P, Cin), _full(w), _full(b)],
        out_specs=_rows(P, Cout),
        out_shape=jax.ShapeDtypeStruct((N, Cout), _f32),
    )(x, w, b)


def _dec_step(enc, itp, w1, w2, b, P):
    N, C1 = enc.shape
    C2 = itp.shape[1]
    Cout = w1.shape[1]

    def body(e_ref, i_ref, w1_ref, w2_ref, b_ref, o_ref):
        y = jnp.dot(e_ref[...], w1_ref[...], preferred_element_type=_f32)
        y = y + jnp.dot(i_ref[...], w2_ref[...], preferred_element_type=_f32)
        o_ref[...] = _leaky(y + b_ref[...])

    return pl.pallas_call(
        body,
        grid=(N // P,),
        in_specs=[_rows(P, C1), _rows(P, C2), _full(w1), _full(w2), _full(b)],
        out_specs=_rows(P, Cout),
        out_shape=jax.ShapeDtypeStruct((N, Cout), _f32),
    )(enc, itp, w1, w2, b)


def _fc_head(x, w1, b1, w2, b2, w3, b3, P):
    N = x.shape[0]
    Cout = w3.shape[1]

    def body(x_ref, w1r, b1r, w2r, b2r, w3r, b3r, o_ref):
        h = _leaky(jnp.dot(x_ref[...], w1r[...],
                           preferred_element_type=_f32) + b1r[...])
        h = _leaky(jnp.dot(h, w2r[...], preferred_element_type=_f32) + b2r[...])
        y = jnp.dot(h, w3r[...], preferred_element_type=_f32) + b3r[...]
        o_ref[...] = y.T

    return pl.pallas_call(
        body,
        grid=(N // P,),
        in_specs=[_rows(P, x.shape[1]), _full(w1), _full(b1), _full(w2),
                  _full(b2), _full(w3), _full(b3)],
        out_specs=pl.BlockSpec((Cout, P), lambda i: (0, i)),
        out_shape=jax.ShapeDtypeStruct((Cout, N), _f32),
    )(x, w1, b1, w2, b2, w3, b3)


_P_ATT = [256, 128, 128, 64]


def kernel(features, xyz_0, xyz_1, xyz_2, xyz_3, neigh_idx_0, neigh_idx_1,
           neigh_idx_2, neigh_idx_3, sub_idx_0, sub_idx_1, sub_idx_2,
           sub_idx_3, interp_idx_0, interp_idx_1, interp_idx_2, interp_idx_3,
           params):
    xyzs = [x[0] for x in (xyz_0, xyz_1, xyz_2, xyz_3)]
    neighs = [n[0].T.reshape(-1) for n in
              (neigh_idx_0, neigh_idx_1, neigh_idx_2, neigh_idx_3)]
    subs = [s[0].T.reshape(-1) for s in
            (sub_idx_0, sub_idx_1, sub_idx_2, sub_idx_3)]
    interps = [t[0, :, 0] for t in
               (interp_idx_0, interp_idx_1, interp_idx_2, interp_idx_3)]

    g0 = params["bn0_gamma"]
    preW = params["fc0W"] * g0[None, :]
    preb = (params["fc0b"] * g0 + params["bn0_beta"])[None, :]

    feature = features[0]
    enc_feats = []
    for i in range(4):
        p = params["enc"][i]
        w1, b1 = _fold(p["mlp1"])
        ws, bs = _fold(p["shortcut"])
        wx1, bx1 = _fold(p["xyz1"])
        wx2, bx2 = _fold(p["xyz2"])
        d2w = w1.shape[1]
        dgw = _pad16(d2w + 4)
        w0 = wx1[0:1]
        wtile = wx1[1:4] + wx1[4:7]
        wga = jnp.zeros((dgw, d2w), _f32).at[d2w:d2w + 3].set(
            wx1[7:10] - wx1[1:4])
        wm1, bm1 = _fold(p["att1"]["mlp"])
        wm2, bm2 = _fold(p["att2"]["mlp"])
        w2, b2 = _fold(p["mlp2"])
        d2 = w1.shape[1]
        P = _P_ATT[i]

        N = feature.shape[0] if i else features.shape[1]
        T, sc = _enc_pre(feature, xyzs[i], w1, b1, ws, bs,
                         (preW, preb) if i == 0 else None, P=256)
        G1 = _gather_rows(T, neighs[i], slabs=_KN)
        aggT, fxyz2 = _att_stage1(G1, xyzs[i], w0, wtile, wga, bx1,
                                  p["att1"]["fcW"], wm1, bm1, wx2, bx2, d2, P)
        G2 = _gather_rows(aggT, neighs[i], slabs=_KN)
        f_enc = _att_stage2(G2, fxyz2, p["att2"]["fcW"], wm2, bm2, w2, b2,
                            sc, d2, P)
        if i == 0:
            enc_feats.append(f_enc)
        Gs = _gather_rows(f_enc, subs[i], slabs=_KN)
        f_s = _pool_max(Gs, min(256, Gs.shape[1]))
        enc_feats.append(f_s)
        feature = f_s

    wd0, bd0 = _fold(params["decoder_0"])
    feature = _conv(feature, wd0, bd0, min(256, feature.shape[0]))
    for j in range(4):
        Gi = _gather_rows(feature, interps[3 - j])
        enc = enc_feats[-j - 2]
        wj, bj = _fold(params["dec"][j])
        c1 = enc.shape[1]
        feature = _dec_step(enc, Gi, wj[:c1], wj[c1:], bj, 256)

    w_fc1, b_fc1 = _fold(params["fc1"])
    w_fc2, b_fc2 = _fold(params["fc2"])
    out = _fc_head(feature, w_fc1, b_fc1, w_fc2, b_fc2,
                   params["fc3W"].T, params["fc3b"][None, :], 512)
    return out[None]


# speedup vs baseline: 1.0212x; 1.0057x over previous
import functools

import jax
import jax.numpy as jnp
from jax import lax
from jax.experimental import pallas as pl
from jax.experimental.pallas import tpu as pltpu
from jax.experimental.pallas import tpu_sc as plsc

_f32 = jnp.float32
_KN = 16


def _pad16(c):
    return -(-c // 16) * 16


def _leaky(x):
    return jnp.where(x >= 0, x, 0.2 * x)


def _fold(p):
    W = p["W"] * p["gamma"][:, None]
    b = p["gamma"] * p["b"] + p["beta"]
    return W.T, b[None, :]


def _gather_rows(table, idx, slabs=None):
    N, D = table.shape
    M = idx.shape[0]
    info = plsc.get_sparse_core_info()
    nc, ns = info.num_cores, info.num_subcores
    nw = nc * ns
    assert M % nw == 0 and (M // nw) % 8 == 0, (M, nw)
    per_w = M // nw
    ch = per_w
    while ch * (D + 1) * 4 > 170000 and ch > 8:
        ch //= 2
    iters = per_w // ch
    rows = M if slabs is None else M // slabs
    out_t = jax.ShapeDtypeStruct((M, D) if slabs is None
                                 else (slabs, rows, D), _f32)
    mesh = plsc.VectorSubcoreMesh(core_axis_name="c", subcore_axis_name="s")

    def body(table_hbm, idx_hbm, out_hbm, i0, i1, r0, r1, s0, s1):
        wid = lax.axis_index("s") * nc + lax.axis_index("c")
        base = wid * per_w
        ib, rb, sb = (i0, i1), (r0, r1), (s0, s1)
        hs = [None, None]

        def write(src, off):
            if slabs is None:
                pltpu.sync_copy(src, out_hbm.at[pl.ds(off, ch)])
            else:
                pltpu.sync_copy(src,
                                out_hbm.at[off // rows, pl.ds(off % rows, ch)])

        for it in range(iters):
            b = it & 1
            off = base + it * ch
            pltpu.sync_copy(idx_hbm.at[pl.ds(off, ch)], ib[b])
            hs[b] = pltpu.async_copy(table_hbm.at[ib[b]], rb[b], sb[b])
            if it >= 1:
                hs[1 - b].wait()
                write(rb[1 - b], base + (it - 1) * ch)
        last = iters - 1
        hs[last & 1].wait()
        write(rb[last & 1], base + last * ch)

    run = pl.kernel(
        body,
        out_type=out_t,
        mesh=mesh,
        scratch_types=[
            pltpu.VMEM((ch,), jnp.int32),
            pltpu.VMEM((ch,), jnp.int32),
            pltpu.VMEM((ch, D), _f32),
            pltpu.VMEM((ch, D), _f32),
            pltpu.SemaphoreType.DMA,
            pltpu.SemaphoreType.DMA,
        ],
        compiler_params=pltpu.CompilerParams(use_tc_tiling_on_sc=False),
    )
    return run(table, idx)


def _full(w):
    return pl.BlockSpec(w.shape, lambda i: (0,) * w.ndim)


def _rows(P, C):
    return pl.BlockSpec((P, C), lambda i: (i, 0))


def _enc_pre(feature, xyz, w1, b1, ws, bs, pre, P):
    N, Cin = feature.shape
    d2 = w1.shape[1]
    C2 = ws.shape[1]
    Dg = _pad16(d2 + 4)
    pad = Dg - d2 - 4

    def body(x_ref, xyz_ref, *refs):
        if pre is not None:
            pw_ref, pb_ref = refs[0], refs[1]
            wrefs = refs[2:6]
            t_ref, sc_ref = refs[6], refs[7]
            x = jnp.dot(x_ref[...], pw_ref[...],
                        preferred_element_type=_f32) + pb_ref[...]
        else:
            wrefs = refs[0:4]
            t_ref, sc_ref = refs[4], refs[5]
            x = x_ref[...]
        w1_ref, b1_ref, ws_ref, bs_ref = wrefs
        fpc = _leaky(jnp.dot(x, w1_ref[...], preferred_element_type=_f32)
                     + b1_ref[...])
        sc_ref[...] = jnp.dot(x, ws_ref[...],
                              preferred_element_type=_f32) + bs_ref[...]
        xyzb = xyz_ref[...]
        n2 = jnp.sum(xyzb * xyzb, axis=1, keepdims=True)
        parts = [fpc, xyzb, n2]
        if pad:
            parts.append(jnp.zeros((fpc.shape[0], pad), _f32))
        t_ref[...] = jnp.concatenate(parts, axis=1)

    ins = [feature, xyz]
    in_specs = [_rows(P, Cin), _rows(P, 3)]
    if pre is not None:
        ins += [pre[0], pre[1]]
        in_specs += [_full(pre[0]), _full(pre[1])]
    ins += [w1, b1, ws, bs]
    in_specs += [_full(w1), _full(b1), _full(ws), _full(bs)]

    return pl.pallas_call(
        body,
        grid=(N // P,),
        in_specs=in_specs,
        out_specs=[_rows(P, Dg), _rows(P, C2)],
        out_shape=[jax.ShapeDtypeStruct((N, Dg), _f32),
                   jax.ShapeDtypeStruct((N, C2), _f32)],
    )(*ins)


def _kblock(Kn, P, C):
    return pl.BlockSpec((Kn, P, C), lambda i: (0, i, 0))


def _fold16(y, op):
    a = op(y[:8], y[8:])
    a = op(a[:4], a[4:])
    a = op(a[:2], a[2:])
    return op(a[0], a[1])


def _att_agg(fcat, fcW, P, d):
    att = jnp.dot(fcat, fcW, preferred_element_type=_f32)
    e = jnp.exp(att)
    y = jnp.concatenate([e, fcat * e], axis=1).reshape(_KN, P, 2 * d)
    y0 = _fold16(y, jnp.add)
    return y0[:, d:] / y0[:, :d]


def _att_stage1(g1, xyz, w0, wtile, wga, bx1, fcW, wm, bm, wx2, bx2, d2, P):
    Kn, N, Dg = g1.shape
    d = 2 * d2
    PK = P * _KN
    d2p = _pad16(d2)

    def body(g_ref, xyz_ref, w0_ref, wt_ref, wga_ref, bx1_ref, fcw_ref,
             wm_ref, bm_ref, wx2_ref, bx2_ref, agg_ref, fxyz2_ref):
        gf = g_ref[...].reshape(PK, Dg)
        xyzb = xyz_ref[...]
        n2i = jnp.sum(xyzb * xyzb, axis=1, keepdims=True)
        bpt = jnp.dot(xyzb, wt_ref[...],
                      preferred_element_type=_f32) + bx1_ref[...]
        v4 = jnp.concatenate([-2.0 * xyzb, jnp.ones((P, 1), _f32)], axis=1)
        u4 = gf[:, d2:d2 + 4]
        m = u4 * jnp.broadcast_to(v4[None], (Kn, P, 4)).reshape(PK, 4)
        dis2 = (jnp.dot(m, jnp.ones((4, 1), _f32),
                        preferred_element_type=_f32)
                + jnp.broadcast_to(n2i[None], (Kn, P, 1)).reshape(PK, 1))
        dis = jnp.sqrt(jnp.maximum(dis2, 0.0) + 1e-12)
        f_xyz = _leaky(dis * w0_ref[...]
                       + jnp.dot(gf, wga_ref[...], preferred_element_type=_f32)
                       + jnp.broadcast_to(bpt[None], (Kn, P, d2)
                                          ).reshape(PK, d2))
        fcat = jnp.concatenate([gf[:, :d2], f_xyz], axis=1)
        agg = _att_agg(fcat, fcw_ref[...], P, d)
        f_agg = _leaky(jnp.dot(agg, wm_ref[...],
                               preferred_element_type=_f32) + bm_ref[...])
        if d2p > d2:
            f_agg = jnp.concatenate(
                [f_agg, jnp.zeros((P, d2p - d2), _f32)], axis=1)
        agg_ref[...] = f_agg
        fxyz2_ref[...] = _leaky(jnp.dot(f_xyz, wx2_ref[...],
                                        preferred_element_type=_f32)
                                + bx2_ref[...]).reshape(Kn, P, d2)

    return pl.pallas_call(
        body,
        grid=(N // P,),
        in_specs=[_kblock(Kn, P, Dg), _rows(P, 3), _full(w0), _full(wtile),
                  _full(wga), _full(bx1), _full(fcW), _full(wm), _full(bm),
                  _full(wx2), _full(bx2)],
        out_specs=[_rows(P, d2p), _kblock(Kn, P, d2)],
        out_shape=[jax.ShapeDtypeStruct((N, d2p), _f32),
                   jax.ShapeDtypeStruct((Kn, N, d2), _f32)],
    )(g1, xyz, w0, wtile, wga, bx1, fcW, wm, bm, wx2, bx2)


def _att_stage2(g2, fxyz2, fcW, wm, bm, w2, b2, sc, d2, P):
    N = sc.shape[0]
    d = 2 * d2
    C2 = sc.shape[1]
    PK = P * _KN
    Kn = g2.shape[0]
    d2p = g2.shape[2]

    def body(g_ref, fx_ref, fcw_ref, wm_ref, bm_ref, w2_ref, b2_ref,
             sc_ref, o_ref):
        f_neigh = g_ref[...][:, :, :d2].reshape(PK, d2)
        fcat = jnp.concatenate([f_neigh, fx_ref[...].reshape(PK, d2)], axis=1)
        agg = _att_agg(fcat, fcw_ref[...], P, d)
        f = _leaky(jnp.dot(agg, wm_ref[...],
                           preferred_element_type=_f32) + bm_ref[...])
        fpc = jnp.dot(f, w2_ref[...], preferred_element_type=_f32) + b2_ref[...]
        o_ref[...] = _leaky(fpc + sc_ref[...])

    return pl.pallas_call(
        body,
        grid=(N // P,),
        in_specs=[_kblock(Kn, P, d2p), _kblock(Kn, P, d2), _full(fcW),
                  _full(wm), _full(bm), _full(w2), _full(b2), _rows(P, C2)],
        out_specs=_rows(P, C2),
        out_shape=jax.ShapeDtypeStruct((N, C2), _f32),
    )(g2, fxyz2, fcW, wm, bm, w2, b2, sc)


def _pool_max(g3, P):
    Kn, N2, C = g3.shape

    def body(g_ref, o_ref):
        o_ref[...] = _fold16(g_ref[...], jnp.maximum)

    return pl.pallas_call(
        body,
        grid=(N2 // P,),
        in_specs=[_kblock(Kn, P, C)],
        out_specs=_rows(P, C),
        out_shape=jax.ShapeDtypeStruct((N2, C), _f32),
    )(g3)


def _conv(x, w, b, P):
    N, Cin = x.shape
    Cout = w.shape[1]

    def body(x_ref, w_ref, b_ref, o_ref):
        o_ref[...] = _leaky(jnp.dot(x_ref[...], w_ref[...],
                                    preferred_element_type=_f32) + b_ref[...])

    return pl.pallas_call(
        body,
        grid=(N // P,),
        in_specs=[_rows(P, Cin), _full(w), _full(b)],
        out_specs=_rows(P, Cout),
        out_shape=jax.ShapeDtypeStruct((N, Cout), _f32),
    )(x, w, b)


def _dec_step(enc, itp, w1, w2, b, P):
    N, C1 = enc.shape
    C2 = itp.shape[1]
    Cout = w1.shape[1]

    def body(e_ref, i_ref, w1_ref, w2_ref, b_ref, o_ref):
        y = jnp.dot(e_ref[...], w1_ref[...], preferred_element_type=_f32)
        y = y + jnp.dot(i_ref[...], w2_ref[...], preferred_element_type=_f32)
        o_ref[...] = _leaky(y + b_ref[...])

    return pl.pallas_call(
        body,
        grid=(N // P,),
        in_specs=[_rows(P, C1), _rows(P, C2), _full(w1), _full(w2), _full(b)],
        out_specs=_rows(P, Cout),
        out_shape=jax.ShapeDtypeStruct((N, Cout), _f32),
    )(enc, itp, w1, w2, b)


def _fc_head(x, w1, b1, w2, b2, w3, b3, P):
    N = x.shape[0]
    Cout = w3.shape[1]

    def body(x_ref, w1r, b1r, w2r, b2r, w3r, b3r, o_ref):
        h = _leaky(jnp.dot(x_ref[...], w1r[...],
                           preferred_element_type=_f32) + b1r[...])
        h = _leaky(jnp.dot(h, w2r[...], preferred_element_type=_f32) + b2r[...])
        y = jnp.dot(h, w3r[...], preferred_element_type=_f32) + b3r[...]
        o_ref[...] = y.T

    return pl.pallas_call(
        body,
        grid=(N // P,),
        in_specs=[_rows(P, x.shape[1]), _full(w1), _full(b1), _full(w2),
                  _full(b2), _full(w3), _full(b3)],
        out_specs=pl.BlockSpec((Cout, P), lambda i: (0, i)),
        out_shape=jax.ShapeDtypeStruct((Cout, N), _f32),
    )(x, w1, b1, w2, b2, w3, b3)


_P_ATT = [256, 128, 128, 64]


def kernel(features, xyz_0, xyz_1, xyz_2, xyz_3, neigh_idx_0, neigh_idx_1,
           neigh_idx_2, neigh_idx_3, sub_idx_0, sub_idx_1, sub_idx_2,
           sub_idx_3, interp_idx_0, interp_idx_1, interp_idx_2, interp_idx_3,
           params):
    xyzs = [x[0] for x in (xyz_0, xyz_1, xyz_2, xyz_3)]
    neighs = [n[0].T.reshape(-1) for n in
              (neigh_idx_0, neigh_idx_1, neigh_idx_2, neigh_idx_3)]
    subs = [s[0].T.reshape(-1) for s in
            (sub_idx_0, sub_idx_1, sub_idx_2, sub_idx_3)]
    interps = [t[0, :, 0] for t in
               (interp_idx_0, interp_idx_1, interp_idx_2, interp_idx_3)]

    g0 = params["bn0_gamma"]
    preW = params["fc0W"] * g0[None, :]
    preb = (params["fc0b"] * g0 + params["bn0_beta"])[None, :]

    feature = features[0]
    enc_feats = []
    for i in range(4):
        p = params["enc"][i]
        w1, b1 = _fold(p["mlp1"])
        ws, bs = _fold(p["shortcut"])
        wx1, bx1 = _fold(p["xyz1"])
        wx2, bx2 = _fold(p["xyz2"])
        d2w = w1.shape[1]
        dgw = _pad16(d2w + 4)
        w0 = wx1[0:1]
        wtile = wx1[1:4] + wx1[4:7]
        wga = jnp.zeros((dgw, d2w), _f32).at[d2w:d2w + 3].set(
            wx1[7:10] - wx1[1:4])
        wm1, bm1 = _fold(p["att1"]["mlp"])
        wm2, bm2 = _fold(p["att2"]["mlp"])
        w2, b2 = _fold(p["mlp2"])
        d2 = w1.shape[1]
        P = _P_ATT[i]

        N = feature.shape[0] if i else features.shape[1]
        T, sc = _enc_pre(feature, xyzs[i], w1, b1, ws, bs,
                         (preW, preb) if i == 0 else None, P=256)
        G1 = _gather_rows(T, neighs[i], slabs=_KN)
        aggT, fxyz2 = _att_stage1(G1, xyzs[i], w0, wtile, wga, bx1,
                                  p["att1"]["fcW"], wm1, bm1, wx2, bx2, d2, P)
        G2 = _gather_rows(aggT, neighs[i], slabs=_KN)
        f_enc = _att_stage2(G2, fxyz2, p["att2"]["fcW"], wm2, bm2, w2, b2,
                            sc, d2, P)
        if i == 0:
            enc_feats.append(f_enc)
        Gs = _gather_rows(f_enc, subs[i], slabs=_KN)
        f_s = _pool_max(Gs, min(256, Gs.shape[1]))
        enc_feats.append(f_s)
        feature = f_s

    wd0, bd0 = _fold(params["decoder_0"])
    feature = _conv(feature, wd0, bd0, min(256, feature.shape[0]))
    for j in range(4):
        Gi = _gather_rows(feature, interps[3 - j])
        enc = enc_feats[-j - 2]
        wj, bj = _fold(params["dec"][j])
        c1 = enc.shape[1]
        feature = _dec_step(enc, Gi, wj[:c1], wj[c1:], bj, 256)

    w_fc1, b_fc1 = _fold(params["fc1"])
    w_fc2, b_fc2 = _fold(params["fc2"])
    out = _fc_head(feature, w_fc1, b_fc1, w_fc2, b_fc2,
                   params["fc3W"].T, params["fc3b"][None, :], 512)
    return out[None]


# speedup vs baseline: 1.0841x; 1.0616x over previous
import functools

import jax
import jax.numpy as jnp
from jax import lax
from jax.experimental import pallas as pl
from jax.experimental.pallas import tpu as pltpu
from jax.experimental.pallas import tpu_sc as plsc

_f32 = jnp.float32
_KN = 16


def _pad16(c):
    return -(-c // 16) * 16


def _leaky(x):
    return jnp.where(x >= 0, x, 0.2 * x)


def _fold(p):
    W = p["W"] * p["gamma"][:, None]
    b = p["gamma"] * p["b"] + p["beta"]
    return W.T, b[None, :]


def _gather_rows(table, idx, slabs=None):
    N, D = table.shape
    M = idx.shape[0]
    info = plsc.get_sparse_core_info()
    nc, ns = info.num_cores, info.num_subcores
    nw = nc * ns
    assert M % nw == 0 and (M // nw) % 8 == 0, (M, nw)
    per_w = M // nw
    ch = per_w
    while ch * (D + 1) * 4 > 170000 and ch > 8:
        ch //= 2
    iters = per_w // ch
    rows = M if slabs is None else M // slabs
    out_t = jax.ShapeDtypeStruct((M, D) if slabs is None
                                 else (slabs, rows, D), _f32)
    mesh = plsc.VectorSubcoreMesh(core_axis_name="c", subcore_axis_name="s")

    def body(table_hbm, idx_hbm, out_hbm, i0, i1, r0, r1, s0, s1):
        wid = lax.axis_index("s") * nc + lax.axis_index("c")
        base = wid * per_w
        ib, rb, sb = (i0, i1), (r0, r1), (s0, s1)
        hs = [None, None]

        def write(src, off):
            if slabs is None:
                pltpu.sync_copy(src, out_hbm.at[pl.ds(off, ch)])
            else:
                pltpu.sync_copy(src,
                                out_hbm.at[off // rows, pl.ds(off % rows, ch)])

        for it in range(iters):
            b = it & 1
            off = base + it * ch
            pltpu.sync_copy(idx_hbm.at[pl.ds(off, ch)], ib[b])
            hs[b] = pltpu.async_copy(table_hbm.at[ib[b]], rb[b], sb[b])
            if it >= 1:
                hs[1 - b].wait()
                write(rb[1 - b], base + (it - 1) * ch)
        last = iters - 1
        hs[last & 1].wait()
        write(rb[last & 1], base + last * ch)

    run = pl.kernel(
        body,
        out_type=out_t,
        mesh=mesh,
        scratch_types=[
            pltpu.VMEM((ch,), jnp.int32),
            pltpu.VMEM((ch,), jnp.int32),
            pltpu.VMEM((ch, D), _f32),
            pltpu.VMEM((ch, D), _f32),
            pltpu.SemaphoreType.DMA,
            pltpu.SemaphoreType.DMA,
        ],
        compiler_params=pltpu.CompilerParams(use_tc_tiling_on_sc=False),
    )
    return run(table, idx)


def _full(w):
    return pl.BlockSpec(w.shape, lambda i: (0,) * w.ndim)


def _rows(P, C):
    return pl.BlockSpec((P, C), lambda i: (i, 0))


def _enc_pre(feature, xyz, w1, b1, ws, bs, pre, P):
    N, Cin = feature.shape
    d2 = w1.shape[1]
    C2 = ws.shape[1]
    Dg = _pad16(d2 + 4)
    pad = Dg - d2 - 4

    def body(x_ref, xyz_ref, *refs):
        if pre is not None:
            pw_ref, pb_ref = refs[0], refs[1]
            wrefs = refs[2:6]
            t_ref, sc_ref = refs[6], refs[7]
            x = jnp.dot(x_ref[...], pw_ref[...],
                        preferred_element_type=_f32) + pb_ref[...]
        else:
            wrefs = refs[0:4]
            t_ref, sc_ref = refs[4], refs[5]
            x = x_ref[...]
        w1_ref, b1_ref, ws_ref, bs_ref = wrefs
        fpc = _leaky(jnp.dot(x, w1_ref[...], preferred_element_type=_f32)
                     + b1_ref[...])
        sc_ref[...] = jnp.dot(x, ws_ref[...],
                              preferred_element_type=_f32) + bs_ref[...]
        xyzb = xyz_ref[...]
        n2 = jnp.sum(xyzb * xyzb, axis=1, keepdims=True)
        parts = [fpc, xyzb, n2]
        if pad:
            parts.append(jnp.zeros((fpc.shape[0], pad), _f32))
        t_ref[...] = jnp.concatenate(parts, axis=1)

    ins = [feature, xyz]
    in_specs = [_rows(P, Cin), _rows(P, 3)]
    if pre is not None:
        ins += [pre[0], pre[1]]
        in_specs += [_full(pre[0]), _full(pre[1])]
    ins += [w1, b1, ws, bs]
    in_specs += [_full(w1), _full(b1), _full(ws), _full(bs)]

    return pl.pallas_call(
        body,
        grid=(N // P,),
        in_specs=in_specs,
        out_specs=[_rows(P, Dg), _rows(P, C2)],
        out_shape=[jax.ShapeDtypeStruct((N, Dg), _f32),
                   jax.ShapeDtypeStruct((N, C2), _f32)],
    )(*ins)


def _kblock(Kn, P, C):
    return pl.BlockSpec((Kn, P, C), lambda i: (0, i, 0))


def _fold16(y, op):
    a = op(y[:8], y[8:])
    a = op(a[:4], a[4:])
    a = op(a[:2], a[2:])
    return op(a[0], a[1])


def _att_agg(fcat, fcW, P, d):
    att = jnp.dot(fcat, fcW, preferred_element_type=_f32)
    e = jnp.exp(att)
    y = jnp.concatenate([e, fcat * e], axis=1).reshape(_KN, P, 2 * d)
    y0 = _fold16(y, jnp.add)
    return y0[:, d:] / y0[:, :d]


def _att_stage1(g1, xyz, w0, wtile, wga, bx1, fcW, wm, bm, wx2, bx2, d2, P):
    Kn, N, Dg = g1.shape
    d = 2 * d2
    PK = P * _KN
    d2p = _pad16(d2)

    def body(g_ref, xyz_ref, w0_ref, wt_ref, wga_ref, bx1_ref, fcw_ref,
             wm_ref, bm_ref, wx2_ref, bx2_ref, agg_ref, fxyz2_ref):
        gf = g_ref[...].reshape(PK, Dg)
        xyzb = xyz_ref[...]
        n2i = jnp.sum(xyzb * xyzb, axis=1, keepdims=True)
        bpt = jnp.dot(xyzb, wt_ref[...],
                      preferred_element_type=_f32) + bx1_ref[...]
        v4 = jnp.concatenate([-2.0 * xyzb, jnp.ones((P, 1), _f32)], axis=1)
        u4 = gf[:, d2:d2 + 4]
        m = u4 * jnp.broadcast_to(v4[None], (Kn, P, 4)).reshape(PK, 4)
        dis2 = (jnp.dot(m, jnp.ones((4, 1), _f32),
                        preferred_element_type=_f32)
                + jnp.broadcast_to(n2i[None], (Kn, P, 1)).reshape(PK, 1))
        dis = jnp.sqrt(jnp.maximum(dis2, 0.0) + 1e-12)
        f_xyz = _leaky(dis * w0_ref[...]
                       + jnp.dot(gf, wga_ref[...], preferred_element_type=_f32)
                       + jnp.broadcast_to(bpt[None], (Kn, P, d2)
                                          ).reshape(PK, d2))
        fcat = jnp.concatenate([gf[:, :d2], f_xyz], axis=1)
        agg = _att_agg(fcat, fcw_ref[...], P, d)
        f_agg = _leaky(jnp.dot(agg, wm_ref[...],
                               preferred_element_type=_f32) + bm_ref[...])
        if d2p > d2:
            f_agg = jnp.concatenate(
                [f_agg, jnp.zeros((P, d2p - d2), _f32)], axis=1)
        agg_ref[...] = f_agg
        fxyz2_ref[...] = _leaky(jnp.dot(f_xyz, wx2_ref[...],
                                        preferred_element_type=_f32)
                                + bx2_ref[...]).reshape(Kn, P, d2)

    return pl.pallas_call(
        body,
        grid=(N // P,),
        in_specs=[_kblock(Kn, P, Dg), _rows(P, 3), _full(w0), _full(wtile),
                  _full(wga), _full(bx1), _full(fcW), _full(wm), _full(bm),
                  _full(wx2), _full(bx2)],
        out_specs=[_rows(P, d2p), _kblock(Kn, P, d2)],
        out_shape=[jax.ShapeDtypeStruct((N, d2p), _f32),
                   jax.ShapeDtypeStruct((Kn, N, d2), _f32)],
    )(g1, xyz, w0, wtile, wga, bx1, fcW, wm, bm, wx2, bx2)


def _att_stage2(g2, fxyz2, fcW, wm, bm, w2, b2, sc, d2, P):
    N = sc.shape[0]
    d = 2 * d2
    C2 = sc.shape[1]
    PK = P * _KN
    Kn = g2.shape[0]
    d2p = g2.shape[2]

    def body(g_ref, fx_ref, fcw_ref, wm_ref, bm_ref, w2_ref, b2_ref,
             sc_ref, o_ref):
        f_neigh = g_ref[...][:, :, :d2].reshape(PK, d2)
        fcat = jnp.concatenate([f_neigh, fx_ref[...].reshape(PK, d2)], axis=1)
        agg = _att_agg(fcat, fcw_ref[...], P, d)
        f = _leaky(jnp.dot(agg, wm_ref[...],
                           preferred_element_type=_f32) + bm_ref[...])
        fpc = jnp.dot(f, w2_ref[...], preferred_element_type=_f32) + b2_ref[...]
        o_ref[...] = _leaky(fpc + sc_ref[...])

    return pl.pallas_call(
        body,
        grid=(N // P,),
        in_specs=[_kblock(Kn, P, d2p), _kblock(Kn, P, d2), _full(fcW),
                  _full(wm), _full(bm), _full(w2), _full(b2), _rows(P, C2)],
        out_specs=_rows(P, C2),
        out_shape=jax.ShapeDtypeStruct((N, C2), _f32),
    )(g2, fxyz2, fcW, wm, bm, w2, b2, sc)


def _pool_max(g3, P):
    Kn, N2, C = g3.shape

    def body(g_ref, o_ref):
        o_ref[...] = _fold16(g_ref[...], jnp.maximum)

    return pl.pallas_call(
        body,
        grid=(N2 // P,),
        in_specs=[_kblock(Kn, P, C)],
        out_specs=_rows(P, C),
        out_shape=jax.ShapeDtypeStruct((N2, C), _f32),
    )(g3)


def _conv(x, w, b, P):
    N, Cin = x.shape
    Cout = w.shape[1]

    def body(x_ref, w_ref, b_ref, o_ref):
        o_ref[...] = _leaky(jnp.dot(x_ref[...], w_ref[...],
                                    preferred_element_type=_f32) + b_ref[...])

    return pl.pallas_call(
        body,
        grid=(N // P,),
        in_specs=[_rows(P, Cin), _full(w), _full(b)],
        out_specs=_rows(P, Cout),
        out_shape=jax.ShapeDtypeStruct((N, Cout), _f32),
    )(x, w, b)


def _dec_step(enc, itp, w1, w2, b, P):
    N, C1 = enc.shape
    C2 = itp.shape[1]
    Cout = w1.shape[1]

    def body(e_ref, i_ref, w1_ref, w2_ref, b_ref, o_ref):
        y = jnp.dot(e_ref[...], w1_ref[...], preferred_element_type=_f32)
        y = y + jnp.dot(i_ref[...], w2_ref[...], preferred_element_type=_f32)
        o_ref[...] = _leaky(y + b_ref[...])

    return pl.pallas_call(
        body,
        grid=(N // P,),
        in_specs=[_rows(P, C1), _rows(P, C2), _full(w1), _full(w2), _full(b)],
        out_specs=_rows(P, Cout),
        out_shape=jax.ShapeDtypeStruct((N, Cout), _f32),
    )(enc, itp, w1, w2, b)


def _fc_head(x, w1, b1, w2, b2, w3, b3, P):
    N = x.shape[0]
    Cout = w3.shape[1]

    def body(x_ref, w1r, b1r, w2r, b2r, w3r, b3r, o_ref):
        h = _leaky(jnp.dot(x_ref[...], w1r[...],
                           preferred_element_type=_f32) + b1r[...])
        h = _leaky(jnp.dot(h, w2r[...], preferred_element_type=_f32) + b2r[...])
        y = jnp.dot(h, w3r[...], preferred_element_type=_f32) + b3r[...]
        o_ref[...] = y.T

    return pl.pallas_call(
        body,
        grid=(N // P,),
        in_specs=[_rows(P, x.shape[1]), _full(w1), _full(b1), _full(w2),
                  _full(b2), _full(w3), _full(b3)],
        out_specs=pl.BlockSpec((Cout, P), lambda i: (0, i)),
        out_shape=jax.ShapeDtypeStruct((Cout, N), _f32),
    )(x, w1, b1, w2, b2, w3, b3)


_P_ATT = [512, 256, 128, 64]


def kernel(features, xyz_0, xyz_1, xyz_2, xyz_3, neigh_idx_0, neigh_idx_1,
           neigh_idx_2, neigh_idx_3, sub_idx_0, sub_idx_1, sub_idx_2,
           sub_idx_3, interp_idx_0, interp_idx_1, interp_idx_2, interp_idx_3,
           params):
    xyzs = [x[0] for x in (xyz_0, xyz_1, xyz_2, xyz_3)]
    neighs = [n[0].T.reshape(-1) for n in
              (neigh_idx_0, neigh_idx_1, neigh_idx_2, neigh_idx_3)]
    subs = [s[0].T.reshape(-1) for s in
            (sub_idx_0, sub_idx_1, sub_idx_2, sub_idx_3)]
    interps = [t[0, :, 0] for t in
               (interp_idx_0, interp_idx_1, interp_idx_2, interp_idx_3)]

    g0 = params["bn0_gamma"]
    preW = params["fc0W"] * g0[None, :]
    preb = (params["fc0b"] * g0 + params["bn0_beta"])[None, :]

    feature = features[0]
    enc_feats = []
    for i in range(4):
        p = params["enc"][i]
        w1, b1 = _fold(p["mlp1"])
        ws, bs = _fold(p["shortcut"])
        wx1, bx1 = _fold(p["xyz1"])
        wx2, bx2 = _fold(p["xyz2"])
        d2w = w1.shape[1]
        dgw = _pad16(d2w + 4)
        w0 = wx1[0:1]
        wtile = wx1[1:4] + wx1[4:7]
        wga = jnp.zeros((dgw, d2w), _f32).at[d2w:d2w + 3].set(
            wx1[7:10] - wx1[1:4])
        wm1, bm1 = _fold(p["att1"]["mlp"])
        wm2, bm2 = _fold(p["att2"]["mlp"])
        w2, b2 = _fold(p["mlp2"])
        d2 = w1.shape[1]
        P = _P_ATT[i]

        N = feature.shape[0] if i else features.shape[1]
        T, sc = _enc_pre(feature, xyzs[i], w1, b1, ws, bs,
                         (preW, preb) if i == 0 else None, P=256)
        G1 = _gather_rows(T, neighs[i], slabs=_KN)
        aggT, fxyz2 = _att_stage1(G1, xyzs[i], w0, wtile, wga, bx1,
                                  p["att1"]["fcW"], wm1, bm1, wx2, bx2, d2, P)
        G2 = _gather_rows(aggT, neighs[i], slabs=_KN)
        f_enc = _att_stage2(G2, fxyz2, p["att2"]["fcW"], wm2, bm2, w2, b2,
                            sc, d2, P)
        if i == 0:
            enc_feats.append(f_enc)
        Gs = _gather_rows(f_enc, subs[i], slabs=_KN)
        f_s = _pool_max(Gs, min(256, Gs.shape[1]))
        enc_feats.append(f_s)
        feature = f_s

    wd0, bd0 = _fold(params["decoder_0"])
    feature = _conv(feature, wd0, bd0, min(256, feature.shape[0]))
    for j in range(4):
        Gi = _gather_rows(feature, interps[3 - j])
        enc = enc_feats[-j - 2]
        wj, bj = _fold(params["dec"][j])
        c1 = enc.shape[1]
        feature = _dec_step(enc, Gi, wj[:c1], wj[c1:], bj, 256)

    w_fc1, b_fc1 = _fold(params["fc1"])
    w_fc2, b_fc2 = _fold(params["fc2"])
    out = _fc_head(feature, w_fc1, b_fc1, w_fc2, b_fc2,
                   params["fc3W"].T, params["fc3b"][None, :], 512)
    return out[None]


# speedup vs baseline: 1.1964x; 1.1036x over previous
import functools

import jax
import jax.numpy as jnp
from jax import lax
from jax.experimental import pallas as pl
from jax.experimental.pallas import tpu as pltpu
from jax.experimental.pallas import tpu_sc as plsc

_f32 = jnp.float32
_KN = 16


def _pad16(c):
    return -(-c // 16) * 16


def _leaky(x):
    return jnp.where(x >= 0, x, 0.2 * x)


def _fold(p):
    W = p["W"] * p["gamma"][:, None]
    b = p["gamma"] * p["b"] + p["beta"]
    return W.T, b[None, :]


def _gather_rows(table, idx, slabs=None):
    N, D = table.shape
    M = idx.shape[0]
    info = plsc.get_sparse_core_info()
    nc, ns = info.num_cores, info.num_subcores
    nw = nc * ns
    assert M % nw == 0 and (M // nw) % 8 == 0, (M, nw)
    per_w = M // nw
    ch = per_w
    while ch * (D + 1) * 4 > 170000 and ch > 8:
        ch //= 2
    iters = per_w // ch
    rows = M if slabs is None else M // slabs
    out_t = jax.ShapeDtypeStruct((M, D) if slabs is None
                                 else (slabs, rows, D), _f32)
    mesh = plsc.VectorSubcoreMesh(core_axis_name="c", subcore_axis_name="s")

    def body(table_hbm, idx_hbm, out_hbm, i0, i1, r0, r1, s0, s1):
        wid = lax.axis_index("s") * nc + lax.axis_index("c")
        base = wid * per_w
        ib, rb, sb = (i0, i1), (r0, r1), (s0, s1)
        hs = [None, None]

        def write(src, off):
            if slabs is None:
                pltpu.sync_copy(src, out_hbm.at[pl.ds(off, ch)])
            else:
                pltpu.sync_copy(src,
                                out_hbm.at[off // rows, pl.ds(off % rows, ch)])

        for it in range(iters):
            b = it & 1
            off = base + it * ch
            pltpu.sync_copy(idx_hbm.at[pl.ds(off, ch)], ib[b])
            hs[b] = pltpu.async_copy(table_hbm.at[ib[b]], rb[b], sb[b])
            if it >= 1:
                hs[1 - b].wait()
                write(rb[1 - b], base + (it - 1) * ch)
        last = iters - 1
        hs[last & 1].wait()
        write(rb[last & 1], base + last * ch)

    run = pl.kernel(
        body,
        out_type=out_t,
        mesh=mesh,
        scratch_types=[
            pltpu.VMEM((ch,), jnp.int32),
            pltpu.VMEM((ch,), jnp.int32),
            pltpu.VMEM((ch, D), _f32),
            pltpu.VMEM((ch, D), _f32),
            pltpu.SemaphoreType.DMA,
            pltpu.SemaphoreType.DMA,
        ],
        compiler_params=pltpu.CompilerParams(use_tc_tiling_on_sc=False),
    )
    return run(table, idx)


def _full(w):
    return pl.BlockSpec(w.shape, lambda i: (0,) * w.ndim)


def _rows(P, C):
    return pl.BlockSpec((P, C), lambda i: (i, 0))


def _enc_pre(feature, xyz, w1, b1, ws, bs, pre, P):
    N, Cin = feature.shape
    d2 = w1.shape[1]
    C2 = ws.shape[1]
    Dg = _pad16(d2 + 4)
    pad = Dg - d2 - 4

    def body(x_ref, xyz_ref, *refs):
        if pre is not None:
            pw_ref, pb_ref = refs[0], refs[1]
            wrefs = refs[2:6]
            t_ref, sc_ref = refs[6], refs[7]
            x = jnp.dot(x_ref[...], pw_ref[...],
                        preferred_element_type=_f32) + pb_ref[...]
        else:
            wrefs = refs[0:4]
            t_ref, sc_ref = refs[4], refs[5]
            x = x_ref[...]
        w1_ref, b1_ref, ws_ref, bs_ref = wrefs
        fpc = _leaky(jnp.dot(x, w1_ref[...], preferred_element_type=_f32)
                     + b1_ref[...])
        sc_ref[...] = jnp.dot(x, ws_ref[...],
                              preferred_element_type=_f32) + bs_ref[...]
        xyzb = xyz_ref[...]
        n2 = jnp.sum(xyzb * xyzb, axis=1, keepdims=True)
        parts = [fpc, xyzb, n2]
        if pad:
            parts.append(jnp.zeros((fpc.shape[0], pad), _f32))
        t_ref[...] = jnp.concatenate(parts, axis=1)

    ins = [feature, xyz]
    in_specs = [_rows(P, Cin), _rows(P, 3)]
    if pre is not None:
        ins += [pre[0], pre[1]]
        in_specs += [_full(pre[0]), _full(pre[1])]
    ins += [w1, b1, ws, bs]
    in_specs += [_full(w1), _full(b1), _full(ws), _full(bs)]

    return pl.pallas_call(
        body,
        grid=(N // P,),
        in_specs=in_specs,
        out_specs=[_rows(P, Dg), _rows(P, C2)],
        out_shape=[jax.ShapeDtypeStruct((N, Dg), _f32),
                   jax.ShapeDtypeStruct((N, C2), _f32)],
    )(*ins)


def _kblock(Kn, P, C):
    return pl.BlockSpec((Kn, P, C), lambda i: (0, i, 0))


def _fold16(y, op):
    a = op(y[:8], y[8:])
    a = op(a[:4], a[4:])
    a = op(a[:2], a[2:])
    return op(a[0], a[1])


def _att_agg(fcat, fcW, P, d):
    att = jnp.dot(fcat, fcW, preferred_element_type=_f32)
    e = jnp.exp(att)
    y = jnp.concatenate([e, fcat * e], axis=1).reshape(_KN, P, 2 * d)
    y0 = _fold16(y, jnp.add)
    return y0[:, d:] / y0[:, :d]


def _att_stage1(g1, xyz, w0, wtile, wga, bx1, fcW, wm, bm, wx2, bx2, d2, P):
    Kn, N, Dg = g1.shape
    d = 2 * d2
    PK = P * _KN
    d2p = _pad16(d2)

    def body(g_ref, xyz_ref, w0_ref, wt_ref, wga_ref, bx1_ref, fcw_ref,
             wm_ref, bm_ref, wx2_ref, bx2_ref, agg_ref, fxyz2_ref):
        gf = g_ref[...].reshape(PK, Dg)
        xyzb = xyz_ref[...]
        n2i = jnp.sum(xyzb * xyzb, axis=1, keepdims=True)
        bpt = jnp.dot(xyzb, wt_ref[...],
                      preferred_element_type=_f32) + bx1_ref[...]
        v4 = jnp.concatenate([-2.0 * xyzb, jnp.ones((P, 1), _f32)], axis=1)
        u4 = gf[:, d2:d2 + 4]
        m = u4 * jnp.broadcast_to(v4[None], (Kn, P, 4)).reshape(PK, 4)
        dis2 = (jnp.dot(m, jnp.ones((4, 1), _f32),
                        preferred_element_type=_f32)
                + jnp.broadcast_to(n2i[None], (Kn, P, 1)).reshape(PK, 1))
        dis = jnp.sqrt(jnp.maximum(dis2, 0.0) + 1e-12)
        f_xyz = _leaky(dis * w0_ref[...]
                       + jnp.dot(gf, wga_ref[...], preferred_element_type=_f32)
                       + jnp.broadcast_to(bpt[None], (Kn, P, d2)
                                          ).reshape(PK, d2))
        fcat = jnp.concatenate([gf[:, :d2], f_xyz], axis=1)
        agg = _att_agg(fcat, fcw_ref[...], P, d)
        f_agg = _leaky(jnp.dot(agg, wm_ref[...],
                               preferred_element_type=_f32) + bm_ref[...])
        if d2p > d2:
            f_agg = jnp.concatenate(
                [f_agg, jnp.zeros((P, d2p - d2), _f32)], axis=1)
        agg_ref[...] = f_agg
        fxyz2_ref[...] = _leaky(jnp.dot(f_xyz, wx2_ref[...],
                                        preferred_element_type=_f32)
                                + bx2_ref[...]).reshape(Kn, P, d2)

    return pl.pallas_call(
        body,
        grid=(N // P,),
        in_specs=[_kblock(Kn, P, Dg), _rows(P, 3), _full(w0), _full(wtile),
                  _full(wga), _full(bx1), _full(fcW), _full(wm), _full(bm),
                  _full(wx2), _full(bx2)],
        out_specs=[_rows(P, d2p), _kblock(Kn, P, d2)],
        out_shape=[jax.ShapeDtypeStruct((N, d2p), _f32),
                   jax.ShapeDtypeStruct((Kn, N, d2), _f32)],
    )(g1, xyz, w0, wtile, wga, bx1, fcW, wm, bm, wx2, bx2)


def _att_stage2(g2, fxyz2, fcW, wm, bm, w2, b2, sc, d2, P):
    N = sc.shape[0]
    d = 2 * d2
    C2 = sc.shape[1]
    PK = P * _KN
    Kn = g2.shape[0]
    d2p = g2.shape[2]

    def body(g_ref, fx_ref, fcw_ref, wm_ref, bm_ref, w2_ref, b2_ref,
             sc_ref, o_ref):
        f_neigh = g_ref[...][:, :, :d2].reshape(PK, d2)
        fcat = jnp.concatenate([f_neigh, fx_ref[...].reshape(PK, d2)], axis=1)
        agg = _att_agg(fcat, fcw_ref[...], P, d)
        f = _leaky(jnp.dot(agg, wm_ref[...],
                           preferred_element_type=_f32) + bm_ref[...])
        fpc = jnp.dot(f, w2_ref[...], preferred_element_type=_f32) + b2_ref[...]
        o_ref[...] = _leaky(fpc + sc_ref[...])

    return pl.pallas_call(
        body,
        grid=(N // P,),
        in_specs=[_kblock(Kn, P, d2p), _kblock(Kn, P, d2), _full(fcW),
                  _full(wm), _full(bm), _full(w2), _full(b2), _rows(P, C2)],
        out_specs=_rows(P, C2),
        out_shape=jax.ShapeDtypeStruct((N, C2), _f32),
    )(g2, fxyz2, fcW, wm, bm, w2, b2, sc)


def _pool_max(g3, P):
    Kn, N2, C = g3.shape

    def body(g_ref, o_ref):
        o_ref[...] = _fold16(g_ref[...], jnp.maximum)

    return pl.pallas_call(
        body,
        grid=(N2 // P,),
        in_specs=[_kblock(Kn, P, C)],
        out_specs=_rows(P, C),
        out_shape=jax.ShapeDtypeStruct((N2, C), _f32),
    )(g3)


def _conv(x, w, b, P):
    N, Cin = x.shape
    Cout = w.shape[1]

    def body(x_ref, w_ref, b_ref, o_ref):
        o_ref[...] = _leaky(jnp.dot(x_ref[...], w_ref[...],
                                    preferred_element_type=_f32) + b_ref[...])

    return pl.pallas_call(
        body,
        grid=(N // P,),
        in_specs=[_rows(P, Cin), _full(w), _full(b)],
        out_specs=_rows(P, Cout),
        out_shape=jax.ShapeDtypeStruct((N, Cout), _f32),
    )(x, w, b)


def _dec_step(enc, itp, w1, w2, b, P):
    N, C1 = enc.shape
    C2 = itp.shape[1]
    Cout = w1.shape[1]

    def body(e_ref, i_ref, w1_ref, w2_ref, b_ref, o_ref):
        y = jnp.dot(e_ref[...], w1_ref[...], preferred_element_type=_f32)
        y = y + jnp.dot(i_ref[...], w2_ref[...], preferred_element_type=_f32)
        o_ref[...] = _leaky(y + b_ref[...])

    return pl.pallas_call(
        body,
        grid=(N // P,),
        in_specs=[_rows(P, C1), _rows(P, C2), _full(w1), _full(w2), _full(b)],
        out_specs=_rows(P, Cout),
        out_shape=jax.ShapeDtypeStruct((N, Cout), _f32),
    )(enc, itp, w1, w2, b)


def _fc_head(x, w1, b1, w2, b2, w3, b3, P):
    N = x.shape[0]
    Cout = w3.shape[1]

    def body(x_ref, w1r, b1r, w2r, b2r, w3r, b3r, o_ref):
        h = _leaky(jnp.dot(x_ref[...], w1r[...],
                           preferred_element_type=_f32) + b1r[...])
        h = _leaky(jnp.dot(h, w2r[...], preferred_element_type=_f32) + b2r[...])
        y = jnp.dot(h, w3r[...], preferred_element_type=_f32) + b3r[...]
        o_ref[...] = y.T

    return pl.pallas_call(
        body,
        grid=(N // P,),
        in_specs=[_rows(P, x.shape[1]), _full(w1), _full(b1), _full(w2),
                  _full(b2), _full(w3), _full(b3)],
        out_specs=pl.BlockSpec((Cout, P), lambda i: (0, i)),
        out_shape=jax.ShapeDtypeStruct((Cout, N), _f32),
    )(x, w1, b1, w2, b2, w3, b3)


_P_ATT = [1024, 512, 256, 128]


def kernel(features, xyz_0, xyz_1, xyz_2, xyz_3, neigh_idx_0, neigh_idx_1,
           neigh_idx_2, neigh_idx_3, sub_idx_0, sub_idx_1, sub_idx_2,
           sub_idx_3, interp_idx_0, interp_idx_1, interp_idx_2, interp_idx_3,
           params):
    xyzs = [x[0] for x in (xyz_0, xyz_1, xyz_2, xyz_3)]
    neighs = [n[0].T.reshape(-1) for n in
              (neigh_idx_0, neigh_idx_1, neigh_idx_2, neigh_idx_3)]
    subs = [s[0].T.reshape(-1) for s in
            (sub_idx_0, sub_idx_1, sub_idx_2, sub_idx_3)]
    interps = [t[0, :, 0] for t in
               (interp_idx_0, interp_idx_1, interp_idx_2, interp_idx_3)]

    g0 = params["bn0_gamma"]
    preW = params["fc0W"] * g0[None, :]
    preb = (params["fc0b"] * g0 + params["bn0_beta"])[None, :]

    feature = features[0]
    enc_feats = []
    for i in range(4):
        p = params["enc"][i]
        w1, b1 = _fold(p["mlp1"])
        ws, bs = _fold(p["shortcut"])
        wx1, bx1 = _fold(p["xyz1"])
        wx2, bx2 = _fold(p["xyz2"])
        d2w = w1.shape[1]
        dgw = _pad16(d2w + 4)
        w0 = wx1[0:1]
        wtile = wx1[1:4] + wx1[4:7]
        wga = jnp.zeros((dgw, d2w), _f32).at[d2w:d2w + 3].set(
            wx1[7:10] - wx1[1:4])
        wm1, bm1 = _fold(p["att1"]["mlp"])
        wm2, bm2 = _fold(p["att2"]["mlp"])
        w2, b2 = _fold(p["mlp2"])
        d2 = w1.shape[1]
        P = _P_ATT[i]

        N = feature.shape[0] if i else features.shape[1]
        T, sc = _enc_pre(feature, xyzs[i], w1, b1, ws, bs,
                         (preW, preb) if i == 0 else None, P=min(512, N))
        G1 = _gather_rows(T, neighs[i], slabs=_KN)
        aggT, fxyz2 = _att_stage1(G1, xyzs[i], w0, wtile, wga, bx1,
                                  p["att1"]["fcW"], wm1, bm1, wx2, bx2, d2, P)
        G2 = _gather_rows(aggT, neighs[i], slabs=_KN)
        f_enc = _att_stage2(G2, fxyz2, p["att2"]["fcW"], wm2, bm2, w2, b2,
                            sc, d2, P)
        if i == 0:
            enc_feats.append(f_enc)
        Gs = _gather_rows(f_enc, subs[i], slabs=_KN)
        f_s = _pool_max(Gs, min(512, Gs.shape[1]))
        enc_feats.append(f_s)
        feature = f_s

    wd0, bd0 = _fold(params["decoder_0"])
    feature = _conv(feature, wd0, bd0, min(256, feature.shape[0]))
    for j in range(4):
        Gi = _gather_rows(feature, interps[3 - j])
        enc = enc_feats[-j - 2]
        wj, bj = _fold(params["dec"][j])
        c1 = enc.shape[1]
        feature = _dec_step(enc, Gi, wj[:c1], wj[c1:], bj, min(512, Gi.shape[0]))

    w_fc1, b_fc1 = _fold(params["fc1"])
    w_fc2, b_fc2 = _fold(params["fc2"])
    out = _fc_head(feature, w_fc1, b_fc1, w_fc2, b_fc2,
                   params["fc3W"].T, params["fc3b"][None, :], 1024)
    return out[None]
